# baseline (speedup 1.0000x reference)
import jax
import jax.numpy as jnp
from jax.experimental import pallas as pl
from jax.experimental.pallas import tpu as pltpu

_NUM_GRAPHS = 1024


def _pool_kernel(seg_ref, x_ref, o_ref, acc_ref, cnt_ref):
    k = pl.program_id(0)

    seg = seg_ref[0]
    g = o_ref.shape[0]
    gids = jax.lax.broadcasted_iota(jnp.int32, (g, seg.shape[1]), 0)
    eq = gids == seg
    onehot = eq.astype(jnp.bfloat16)

    psum = jnp.dot(onehot, x_ref[...].astype(jnp.bfloat16),
                   preferred_element_type=jnp.float32)
    pcnt = jnp.count_nonzero(eq, axis=1, keepdims=True).astype(jnp.float32)

    @pl.when(k == 0)
    def _():
        acc_ref[...] = psum
        cnt_ref[...] = pcnt

    @pl.when(k != 0)
    def _():
        acc_ref[...] += psum
        cnt_ref[...] += pcnt

    @pl.when(k == pl.num_programs(0) - 1)
    def _():
        cnt = cnt_ref[:, 0:1]
        o_ref[...] = acc_ref[...] * (1.0 / jnp.maximum(cnt, 1.0))


def kernel(x, batch):
    n, d = x.shape
    g = _NUM_GRAPHS
    tn = 16384

    xf = x.astype(jnp.float32)
    segs = batch.astype(jnp.int32)
    n_pad = -(-n // tn) * tn
    if n_pad != n:
        xf = jnp.pad(xf, ((0, n_pad - n), (0, 0)))
        segs = jnp.pad(segs, (0, n_pad - n), constant_values=-1)
    k_tiles = n_pad // tn
    seg3 = segs.reshape(k_tiles, 1, tn)

    out = pl.pallas_call(
        _pool_kernel,
        out_shape=jax.ShapeDtypeStruct((g, d), jnp.float32),
        grid=(k_tiles,),
        in_specs=[
            pl.BlockSpec((1, 1, tn), lambda k: (k, 0, 0)),
            pl.BlockSpec((tn, d), lambda k: (k, 0)),
        ],
        out_specs=pl.BlockSpec((g, d), lambda k: (0, 0)),
        scratch_shapes=[
            pltpu.VMEM((g, d), jnp.float32),
            pltpu.VMEM((g, 1), jnp.float32),
        ],
        compiler_params=pltpu.CompilerParams(
            dimension_semantics=("arbitrary",)),
    )(seg3, xf)
    return out


# speedup vs baseline: 1.0465x; 1.0465x over previous
import jax
import jax.numpy as jnp
from jax.experimental import pallas as pl
from jax.experimental.pallas import tpu as pltpu

_NUM_GRAPHS = 1024


def _pool_kernel(seg_ref, x_ref, o_ref, acc_ref, cnt_ref):
    k = pl.program_id(0)

    seg = seg_ref[0]
    g = o_ref.shape[0]
    gids = jax.lax.broadcasted_iota(jnp.int32, (g, seg.shape[1]), 0)
    eq = gids == seg
    onehot = eq.astype(jnp.bfloat16)

    psum = jnp.dot(onehot, x_ref[...].astype(jnp.bfloat16),
                   preferred_element_type=jnp.float32)
    pcnt = jnp.count_nonzero(eq, axis=1, keepdims=True).astype(jnp.float32)

    @pl.when(k == 0)
    def _():
        acc_ref[...] = psum
        cnt_ref[...] = pcnt

    @pl.when(k != 0)
    def _():
        acc_ref[...] += psum
        cnt_ref[...] += pcnt

    @pl.when(k == pl.num_programs(0) - 1)
    def _():
        cnt = cnt_ref[:, 0:1]
        o_ref[...] = acc_ref[...] * (1.0 / jnp.maximum(cnt, 1.0))


def kernel(x, batch):
    n, d = x.shape
    g = _NUM_GRAPHS
    tn = 4096

    xf = x.astype(jnp.float32)
    segs = batch.astype(jnp.int32)
    n_pad = -(-n // tn) * tn
    if n_pad != n:
        xf = jnp.pad(xf, ((0, n_pad - n), (0, 0)))
        segs = jnp.pad(segs, (0, n_pad - n), constant_values=-1)
    k_tiles = n_pad // tn
    seg3 = segs.reshape(k_tiles, 1, tn)

    out = pl.pallas_call(
        _pool_kernel,
        out_shape=jax.ShapeDtypeStruct((g, d), jnp.float32),
        grid=(k_tiles,),
        in_specs=[
            pl.BlockSpec((1, 1, tn), lambda k: (k, 0, 0)),
            pl.BlockSpec((tn, d), lambda k: (k, 0)),
        ],
        out_specs=pl.BlockSpec((g, d), lambda k: (0, 0)),
        scratch_shapes=[
            pltpu.VMEM((g, d), jnp.float32),
            pltpu.VMEM((g, 1), jnp.float32),
        ],
        compiler_params=pltpu.CompilerParams(
            dimension_semantics=("arbitrary",)),
    )(seg3, xf)
    return out


# speedup vs baseline: 1.0648x; 1.0175x over previous
import jax
import jax.numpy as jnp
from jax.experimental import pallas as pl
from jax.experimental.pallas import tpu as pltpu

_NUM_GRAPHS = 1024


def _pool_kernel(seg_ref, x_ref, o_ref, acc_ref, cnt_ref):
    k = pl.program_id(0)

    seg = seg_ref[0]
    g = o_ref.shape[0]
    gids = jax.lax.broadcasted_iota(jnp.int32, (g, seg.shape[1]), 0)
    eq = gids == seg
    onehot = eq.astype(jnp.bfloat16)

    psum = jnp.dot(onehot, x_ref[...].astype(jnp.bfloat16),
                   preferred_element_type=jnp.float32)
    pcnt = jnp.count_nonzero(eq, axis=1, keepdims=True).astype(jnp.float32)

    @pl.when(k == 0)
    def _():
        acc_ref[...] = psum
        cnt_ref[...] = pcnt

    @pl.when(k != 0)
    def _():
        acc_ref[...] += psum
        cnt_ref[...] += pcnt

    @pl.when(k == pl.num_programs(0) - 1)
    def _():
        cnt = cnt_ref[:, 0:1]
        o_ref[...] = acc_ref[...] * (1.0 / jnp.maximum(cnt, 1.0))


def kernel(x, batch):
    n, d = x.shape
    g = _NUM_GRAPHS
    tn = 8192

    xf = x.astype(jnp.float32)
    segs = batch.astype(jnp.int32)
    n_pad = -(-n // tn) * tn
    if n_pad != n:
        xf = jnp.pad(xf, ((0, n_pad - n), (0, 0)))
        segs = jnp.pad(segs, (0, n_pad - n), constant_values=-1)
    k_tiles = n_pad // tn
    seg3 = segs.reshape(k_tiles, 1, tn)

    out = pl.pallas_call(
        _pool_kernel,
        out_shape=jax.ShapeDtypeStruct((g, d), jnp.float32),
        grid=(k_tiles,),
        in_specs=[
            pl.BlockSpec((1, 1, tn), lambda k: (k, 0, 0)),
            pl.BlockSpec((tn, d), lambda k: (k, 0)),
        ],
        out_specs=pl.BlockSpec((g, d), lambda k: (0, 0)),
        scratch_shapes=[
            pltpu.VMEM((g, d), jnp.float32),
            pltpu.VMEM((g, 1), jnp.float32),
        ],
        compiler_params=pltpu.CompilerParams(
            dimension_semantics=("arbitrary",)),
    )(seg3, xf)
    return out
